# Initial kernel scaffold; baseline (speedup 1.0000x reference)
import functools

import jax
import jax.numpy as jnp
from jax import lax
from jax.experimental import pallas as pl
from jax.experimental.pallas import tpu as pltpu
from jax.experimental.pallas import tpu_sc as plsc

N = 2048
K = 16
BLK = 256
GRID = N // BLK
D1 = 640
DM = 256
FBIG = jnp.float32(3.0e38)
IBIG = jnp.int32(2**30)


def _front_body(cp_ref, f_ref, csp_ref, w1_ref, b1_ref, w2_ref, b2_ref, x_ref):
    h = jnp.dot(cp_ref[...], w1_ref[...], preferred_element_type=jnp.float32)
    h = jnp.maximum(h + b1_ref[...], 0.0)
    h = jnp.dot(h, w2_ref[...], preferred_element_type=jnp.float32)
    h = jnp.maximum(h + b2_ref[...], 0.0)
    x_ref[...] = jnp.concatenate([h, f_ref[...], csp_ref[...]], axis=1)


def _front(Cp, F, CSp, W1p, b1r, W2, b2r):
    return pl.pallas_call(
        _front_body,
        grid=(GRID,),
        in_specs=[
            pl.BlockSpec((BLK, 128), lambda i: (i, 0)),
            pl.BlockSpec((BLK, 512), lambda i: (i, 0)),
            pl.BlockSpec((BLK, 64), lambda i: (i, 0)),
            pl.BlockSpec((128, 64), lambda i: (0, 0)),
            pl.BlockSpec((1, 64), lambda i: (0, 0)),
            pl.BlockSpec((64, 64), lambda i: (0, 0)),
            pl.BlockSpec((1, 64), lambda i: (0, 0)),
        ],
        out_specs=pl.BlockSpec((BLK, D1), lambda i: (i, 0)),
        out_shape=jax.ShapeDtypeStruct((N, D1), jnp.float32),
    )(Cp, F, CSp, W1p, b1r, W2, b2r)


def _pre1_body(x_ref, wd_ref, wb_ref, b_ref, p_ref, q_ref):
    x = x_ref[...]
    p_ref[...] = (
        jnp.dot(x, wd_ref[...], preferred_element_type=jnp.float32) + b_ref[...]
    )
    q_ref[...] = jnp.dot(x, wb_ref[...], preferred_element_type=jnp.float32)


def _pre1(X, Wd, Wb, br, d):
    return pl.pallas_call(
        _pre1_body,
        grid=(GRID,),
        in_specs=[
            pl.BlockSpec((BLK, d), lambda i: (i, 0)),
            pl.BlockSpec((d, DM), lambda i: (0, 0)),
            pl.BlockSpec((d, DM), lambda i: (0, 0)),
            pl.BlockSpec((1, DM), lambda i: (0, 0)),
        ],
        out_specs=[
            pl.BlockSpec((BLK, DM), lambda i: (i, 0)),
            pl.BlockSpec((BLK, DM), lambda i: (i, 0)),
        ],
        out_shape=[
            jax.ShapeDtypeStruct((N, DM), jnp.float32),
            jax.ShapeDtypeStruct((N, DM), jnp.float32),
        ],
    )(X, Wd, Wb, br)


def _pre2_body(pp_ref, mm_ref, wd_ref, wb_ref, b_ref, x_ref, p_ref, q_ref):
    x = jnp.maximum(pp_ref[...] + mm_ref[...], 0.0)
    x_ref[...] = x
    p_ref[...] = (
        jnp.dot(x, wd_ref[...], preferred_element_type=jnp.float32) + b_ref[...]
    )
    q_ref[...] = jnp.dot(x, wb_ref[...], preferred_element_type=jnp.float32)


def _pre2(Pp, Mp, Wd, Wb, br):
    return pl.pallas_call(
        _pre2_body,
        grid=(GRID,),
        in_specs=[
            pl.BlockSpec((BLK, DM), lambda i: (i, 0)),
            pl.BlockSpec((BLK, DM), lambda i: (i, 0)),
            pl.BlockSpec((DM, DM), lambda i: (0, 0)),
            pl.BlockSpec((DM, DM), lambda i: (0, 0)),
            pl.BlockSpec((1, DM), lambda i: (0, 0)),
        ],
        out_specs=[
            pl.BlockSpec((BLK, DM), lambda i: (i, 0)),
            pl.BlockSpec((BLK, DM), lambda i: (i, 0)),
            pl.BlockSpec((BLK, DM), lambda i: (i, 0)),
        ],
        out_shape=[
            jax.ShapeDtypeStruct((N, DM), jnp.float32),
            jax.ShapeDtypeStruct((N, DM), jnp.float32),
            jax.ShapeDtypeStruct((N, DM), jnp.float32),
        ],
    )(Pp, Mp, Wd, Wb, br)


def _pre3_body(pp_ref, mm_ref, r_ref, wd_ref, wb_ref, b_ref, x_ref, p_ref, q_ref):
    x = jnp.maximum(pp_ref[...] + mm_ref[...], 0.0) + r_ref[...]
    x_ref[...] = x
    p_ref[...] = (
        jnp.dot(x, wd_ref[...], preferred_element_type=jnp.float32) + b_ref[...]
    )
    q_ref[...] = jnp.dot(x, wb_ref[...], preferred_element_type=jnp.float32)


def _pre3(Pp, Mp, R, Wd, Wb, br):
    return pl.pallas_call(
        _pre3_body,
        grid=(GRID,),
        in_specs=[
            pl.BlockSpec((BLK, DM), lambda i: (i, 0)),
            pl.BlockSpec((BLK, DM), lambda i: (i, 0)),
            pl.BlockSpec((BLK, DM), lambda i: (i, 0)),
            pl.BlockSpec((DM, DM), lambda i: (0, 0)),
            pl.BlockSpec((DM, DM), lambda i: (0, 0)),
            pl.BlockSpec((1, DM), lambda i: (0, 0)),
        ],
        out_specs=[
            pl.BlockSpec((BLK, DM), lambda i: (i, 0)),
            pl.BlockSpec((BLK, DM), lambda i: (i, 0)),
            pl.BlockSpec((BLK, DM), lambda i: (i, 0)),
        ],
        out_shape=[
            jax.ShapeDtypeStruct((N, DM), jnp.float32),
            jax.ShapeDtypeStruct((N, DM), jnp.float32),
            jax.ShapeDtypeStruct((N, DM), jnp.float32),
        ],
    )(Pp, Mp, R, Wd, Wb, br)


def _knn_body(xq_ref, xt_ref, idx_ref):
    xq = xq_ref[...]
    xt = xt_ref[...]
    inner = jnp.dot(xq, xt, preferred_element_type=jnp.float32)
    sq_q = jnp.sum(xq * xq, axis=1, keepdims=True)
    sq_k = jnp.sum(xt * xt, axis=0, keepdims=True)
    dist = sq_q - 2.0 * inner + sq_k
    iota = lax.broadcasted_iota(jnp.int32, dist.shape, 1)
    cols = []
    for _ in range(K):
        m = jnp.min(dist, axis=1, keepdims=True)
        sel = jnp.where(dist == m, iota, IBIG)
        j = jnp.min(sel, axis=1, keepdims=True)
        cols.append(j)
        dist = jnp.where(iota == j, FBIG, dist)
    idx_ref[...] = jnp.concatenate(cols, axis=1)


def _knn(X, XT, d):
    return pl.pallas_call(
        _knn_body,
        grid=(GRID,),
        in_specs=[
            pl.BlockSpec((BLK, d), lambda i: (i, 0)),
            pl.BlockSpec((d, N), lambda i: (0, 0)),
        ],
        out_specs=pl.BlockSpec((BLK, K), lambda i: (i, 0)),
        out_shape=jax.ShapeDtypeStruct((N, K), jnp.int32),
    )(X, XT)


def _head_body(p3_ref, m3_ref, p2_ref, m2_ref, w6_ref, b6_ref, wot_ref, bo_ref, o_ref):
    x = jnp.maximum(p3_ref[...] + m3_ref[...], 0.0) + jnp.maximum(
        p2_ref[...] + m2_ref[...], 0.0
    )
    h = jnp.dot(x, w6_ref[...], preferred_element_type=jnp.float32)
    h = jnp.maximum(h + b6_ref[...], 0.0)
    o = jnp.sum(h * wot_ref[...], axis=1, keepdims=True) + bo_ref[...]
    o_ref[...] = jnp.maximum(o, 0.0)


def _head(P3, M3, P2, M2, W6, b6r, WoT, bor):
    return pl.pallas_call(
        _head_body,
        grid=(GRID,),
        in_specs=[
            pl.BlockSpec((BLK, DM), lambda i: (i, 0)),
            pl.BlockSpec((BLK, DM), lambda i: (i, 0)),
            pl.BlockSpec((BLK, DM), lambda i: (i, 0)),
            pl.BlockSpec((BLK, DM), lambda i: (i, 0)),
            pl.BlockSpec((DM, 64), lambda i: (0, 0)),
            pl.BlockSpec((1, 64), lambda i: (0, 0)),
            pl.BlockSpec((1, 64), lambda i: (0, 0)),
            pl.BlockSpec((1, 1), lambda i: (0, 0)),
        ],
        out_specs=pl.BlockSpec((BLK, 1), lambda i: (i, 0)),
        out_shape=jax.ShapeDtypeStruct((N, 1), jnp.float32),
    )(P3, M3, P2, M2, W6, b6r, WoT, bor)


_NC = 2
_NS = 16
_NW = _NC * _NS
_PPW = N // _NW
_CH = 8
_NCH = _PPW // _CH
_LANE = 16


def _gathermax_body(q_hbm, idxf_hbm, out_hbm, idx_v, rows_a, rows_b, out_v,
                    sem_a, sem_b):
    wid = lax.axis_index("s") * _NC + lax.axis_index("c")
    base = pl.multiple_of(wid * _PPW, _PPW)
    pltpu.sync_copy(idxf_hbm.at[pl.ds(base * K, _PPW * K)], idx_v)

    bufs = (rows_a, rows_b)
    sems = (sem_a, sem_b)

    def start(ch):
        return pltpu.async_copy(
            q_hbm.at[idx_v.at[pl.ds(ch * _CH * K, _CH * K)]],
            bufs[ch % 2],
            sems[ch % 2],
        )

    cp = start(0)
    for ch in range(_NCH):
        cp.wait()
        if ch + 1 < _NCH:
            cp = start(ch + 1)
        rows = bufs[ch % 2]

        def point_body(p, carry, ch=ch, rows=rows):
            r0 = p * K
            for c in range(DM // _LANE):
                acc = rows[r0, pl.ds(c * _LANE, _LANE)]
                for r in range(1, K):
                    acc = jnp.maximum(acc, rows[r0 + r, pl.ds(c * _LANE, _LANE)])
                out_v[ch * _CH + p, pl.ds(c * _LANE, _LANE)] = acc
            return carry

        lax.fori_loop(0, _CH, point_body, 0)

    pltpu.sync_copy(out_v, out_hbm.at[pl.ds(base, _PPW)])


@functools.partial(
    pl.kernel,
    mesh=plsc.VectorSubcoreMesh(core_axis_name="c", subcore_axis_name="s"),
    out_type=jax.ShapeDtypeStruct((N, DM), jnp.float32),
    scratch_types=[
        pltpu.VMEM((_PPW * K,), jnp.int32),
        pltpu.VMEM((_CH * K, DM), jnp.float32),
        pltpu.VMEM((_CH * K, DM), jnp.float32),
        pltpu.VMEM((_PPW, DM), jnp.float32),
        pltpu.SemaphoreType.DMA,
        pltpu.SemaphoreType.DMA,
    ],
)
def _gathermax(q_hbm, idxf_hbm, out_hbm, idx_v, rows_a, rows_b, out_v,
               sem_a, sem_b):
    _gathermax_body(q_hbm, idxf_hbm, out_hbm, idx_v, rows_a, rows_b, out_v,
                    sem_a, sem_b)


def kernel(Coordinate3D, Feature512D, CenterScore, W1, b1, W2, b2, W3, b3,
           W4, b4, W5, b5, W6, b6, Wo, bo):
    C = Coordinate3D[0]
    F = Feature512D[0]
    CS = CenterScore[0]

    Cp = jnp.pad(C, ((0, 0), (0, 128 - 3)))
    CSp = jnp.pad(CS, ((0, 0), (0, 64 - 1)))
    W1p = jnp.pad(W1, ((0, 128 - 3), (0, 0)))

    d_cat = 577
    W3a, W3b = W3[:d_cat], W3[d_cat:]
    Wd1 = jnp.pad(W3a - W3b, ((0, D1 - d_cat), (0, 0)))
    Wb1 = jnp.pad(W3b, ((0, D1 - d_cat), (0, 0)))
    Wd2, Wb2 = W4[:DM] - W4[DM:], W4[DM:]
    Wd3, Wb3 = W5[:DM] - W5[DM:], W5[DM:]

    b1r = b1.reshape(1, -1)
    b2r = b2.reshape(1, -1)
    b3r = b3.reshape(1, -1)
    b4r = b4.reshape(1, -1)
    b5r = b5.reshape(1, -1)
    b6r = b6.reshape(1, -1)
    WoT = Wo.reshape(1, -1)
    bor = bo.reshape(1, 1)

    X1 = _front(Cp, F, CSp, W1p, b1r, W2, b2r)

    P1, Q1 = _pre1(X1, Wd1, Wb1, b3r, D1)
    idx1 = _knn(X1, X1.T, D1)
    M1 = _gathermax(Q1, idx1.reshape(-1))

    X2, P2, Q2 = _pre2(P1, M1, Wd2, Wb2, b4r)
    idx2 = _knn(X2, X2.T, DM)
    M2 = _gathermax(Q2, idx2.reshape(-1))

    X3, P3, Q3 = _pre3(P2, M2, X2, Wd3, Wb3, b5r)
    idx3 = _knn(X3, X3.T, DM)
    M3 = _gathermax(Q3, idx3.reshape(-1))

    out = _head(P3, M3, P2, M2, W6, b6r, WoT, bor)
    return out.reshape(1, N, 1)


# speedup vs baseline: 6.7948x; 6.7948x over previous
import functools

import jax
import jax.numpy as jnp
from jax import lax
from jax.experimental import pallas as pl
from jax.experimental.pallas import tpu as pltpu
from jax.experimental.pallas import tpu_sc as plsc

N = 2048
K = 16
BLK = 256
GRID = N // BLK
D1 = 640
DM = 256
EBLK = 128
EGRID = N // EBLK
FBIG = 3.0e38
IBIG = 2**30


def _b16(x):
    return x.astype(jnp.bfloat16)


def _front_body(cp_ref, f_ref, csp_ref, w1_ref, b1_ref, w2_ref, b2_ref, x_ref):
    h = jnp.dot(_b16(cp_ref[...]), _b16(w1_ref[...]),
                preferred_element_type=jnp.float32)
    h = jnp.maximum(h + b1_ref[...], 0.0)
    h = jnp.dot(_b16(h), _b16(w2_ref[...]), preferred_element_type=jnp.float32)
    h = jnp.maximum(h + b2_ref[...], 0.0)
    x_ref[...] = jnp.concatenate([h, f_ref[...], csp_ref[...]], axis=1)


def _front(Cp, F, CSp, W1p, b1r, W2, b2r):
    return pl.pallas_call(
        _front_body,
        grid=(GRID,),
        in_specs=[
            pl.BlockSpec((BLK, 128), lambda i: (i, 0)),
            pl.BlockSpec((BLK, 512), lambda i: (i, 0)),
            pl.BlockSpec((BLK, 64), lambda i: (i, 0)),
            pl.BlockSpec((128, 64), lambda i: (0, 0)),
            pl.BlockSpec((1, 64), lambda i: (0, 0)),
            pl.BlockSpec((64, 64), lambda i: (0, 0)),
            pl.BlockSpec((1, 64), lambda i: (0, 0)),
        ],
        out_specs=pl.BlockSpec((BLK, D1), lambda i: (i, 0)),
        out_shape=jax.ShapeDtypeStruct((N, D1), jnp.float32),
    )(Cp, F, CSp, W1p, b1r, W2, b2r)


def _amm_body(x_ref, wa_ref, a_ref):
    a_ref[...] = jnp.dot(_b16(x_ref[...]), _b16(wa_ref[...]),
                         preferred_element_type=jnp.float32)


def _amm(X, Wa, d):
    return pl.pallas_call(
        _amm_body,
        grid=(GRID,),
        in_specs=[
            pl.BlockSpec((BLK, d), lambda i: (i, 0)),
            pl.BlockSpec((d, DM), lambda i: (0, 0)),
        ],
        out_specs=pl.BlockSpec((BLK, DM), lambda i: (i, 0)),
        out_shape=jax.ShapeDtypeStruct((N, DM), jnp.float32),
    )(X, Wa)


def _knn_body(xq_ref, xt_ref, idx_ref):
    xq = xq_ref[...]
    xt = xt_ref[...]
    inner = jnp.dot(_b16(xq), _b16(xt), preferred_element_type=jnp.float32)
    sq_q = jnp.sum(xq * xq, axis=1, keepdims=True)
    sq_k = jnp.sum(xt * xt, axis=0, keepdims=True)
    dist = sq_q - 2.0 * inner + sq_k
    iota = lax.broadcasted_iota(jnp.int32, dist.shape, 1)
    cols = []
    for _ in range(K):
        m = jnp.min(dist, axis=1, keepdims=True)
        sel = jnp.where(dist == m, iota, IBIG)
        j = jnp.min(sel, axis=1, keepdims=True)
        cols.append(j)
        dist = jnp.where(iota == j, FBIG, dist)
    idx_ref[...] = jnp.concatenate(cols, axis=1)


def _knn(X, XT, d):
    return pl.pallas_call(
        _knn_body,
        grid=(GRID,),
        in_specs=[
            pl.BlockSpec((BLK, d), lambda i: (i, 0)),
            pl.BlockSpec((d, N), lambda i: (0, 0)),
        ],
        out_specs=pl.BlockSpec((BLK, K), lambda i: (i, 0)),
        out_shape=jax.ShapeDtypeStruct((N, K), jnp.int32),
    )(X, XT)


def _edge_body(nb_ref, x_ref, a_ref, wb_ref, b_ref, e_ref, *, d):
    nb = nb_ref[...]
    xi = x_ref[...]
    diff = nb.reshape(EBLK, K, d) - xi[:, None, :]
    bm = jnp.dot(_b16(diff.reshape(EBLK * K, d)), _b16(wb_ref[...]),
                 preferred_element_type=jnp.float32)
    bmax = jnp.max(bm.reshape(EBLK, K, DM), axis=1)
    e_ref[...] = jnp.maximum((a_ref[...] + bmax) + b_ref[...], 0.0)


def _edge_res_body(nb_ref, x_ref, a_ref, wb_ref, b_ref, r_ref,
                   e_ref, xn_ref, *, d):
    nb = nb_ref[...]
    xi = x_ref[...]
    diff = nb.reshape(EBLK, K, d) - xi[:, None, :]
    bm = jnp.dot(_b16(diff.reshape(EBLK * K, d)), _b16(wb_ref[...]),
                 preferred_element_type=jnp.float32)
    bmax = jnp.max(bm.reshape(EBLK, K, DM), axis=1)
    e = jnp.maximum((a_ref[...] + bmax) + b_ref[...], 0.0)
    e_ref[...] = e
    xn_ref[...] = e + r_ref[...]


def _edgemax(A, nb, X, Wb, br, d):
    return pl.pallas_call(
        functools.partial(_edge_body, d=d),
        grid=(EGRID,),
        in_specs=[
            pl.BlockSpec((EBLK * K, d), lambda i: (i, 0)),
            pl.BlockSpec((EBLK, d), lambda i: (i, 0)),
            pl.BlockSpec((EBLK, DM), lambda i: (i, 0)),
            pl.BlockSpec((d, DM), lambda i: (0, 0)),
            pl.BlockSpec((1, DM), lambda i: (0, 0)),
        ],
        out_specs=pl.BlockSpec((EBLK, DM), lambda i: (i, 0)),
        out_shape=jax.ShapeDtypeStruct((N, DM), jnp.float32),
    )(nb, X, A, Wb, br)


def _edgemax_res(A, nb, X, Wb, br, R, d):
    return pl.pallas_call(
        functools.partial(_edge_res_body, d=d),
        grid=(EGRID,),
        in_specs=[
            pl.BlockSpec((EBLK * K, d), lambda i: (i, 0)),
            pl.BlockSpec((EBLK, d), lambda i: (i, 0)),
            pl.BlockSpec((EBLK, DM), lambda i: (i, 0)),
            pl.BlockSpec((d, DM), lambda i: (0, 0)),
            pl.BlockSpec((1, DM), lambda i: (0, 0)),
            pl.BlockSpec((EBLK, DM), lambda i: (i, 0)),
        ],
        out_specs=[
            pl.BlockSpec((EBLK, DM), lambda i: (i, 0)),
            pl.BlockSpec((EBLK, DM), lambda i: (i, 0)),
        ],
        out_shape=[
            jax.ShapeDtypeStruct((N, DM), jnp.float32),
            jax.ShapeDtypeStruct((N, DM), jnp.float32),
        ],
    )(nb, X, A, Wb, br, R)


def _head_body(x_ref, w6_ref, b6_ref, wot_ref, bo_ref, o_ref):
    h = jnp.dot(_b16(x_ref[...]), _b16(w6_ref[...]),
                preferred_element_type=jnp.float32)
    h = jnp.maximum(h + b6_ref[...], 0.0)
    hb = _b16(h).astype(jnp.float32)
    wb = _b16(wot_ref[...]).astype(jnp.float32)
    o = jnp.sum(hb * wb, axis=1, keepdims=True) + bo_ref[...]
    o_ref[...] = jnp.maximum(o, 0.0)


def _head(X4, W6, b6r, WoT, bor):
    return pl.pallas_call(
        _head_body,
        grid=(GRID,),
        in_specs=[
            pl.BlockSpec((BLK, DM), lambda i: (i, 0)),
            pl.BlockSpec((DM, 64), lambda i: (0, 0)),
            pl.BlockSpec((1, 64), lambda i: (0, 0)),
            pl.BlockSpec((1, 64), lambda i: (0, 0)),
            pl.BlockSpec((1, 1), lambda i: (0, 0)),
        ],
        out_specs=pl.BlockSpec((BLK, 1), lambda i: (i, 0)),
        out_shape=jax.ShapeDtypeStruct((N, 1), jnp.float32),
    )(X4, W6, b6r, WoT, bor)


_NC = 2
_NS = 16
_NW = _NC * _NS
_PPW = N // _NW
_RPW = _PPW * K


def _make_scgather(d, rc):
    nch = _RPW // rc

    def body(x_hbm, idxf_hbm, out_hbm, idx_v, rows_a, rows_b, sem_a, sem_b):
        wid = lax.axis_index("s") * _NC + lax.axis_index("c")
        r0 = pl.multiple_of(wid * _RPW, _RPW)
        pltpu.sync_copy(idxf_hbm.at[pl.ds(r0, _RPW)], idx_v)

        bufs = (rows_a, rows_b)
        sems = (sem_a, sem_b)

        def start(ch):
            return pltpu.async_copy(
                x_hbm.at[idx_v.at[pl.ds(ch * rc, rc)]],
                bufs[ch % 2],
                sems[ch % 2],
            )

        cp = start(0)
        for ch in range(nch):
            cp.wait()
            if ch + 1 < nch:
                cp = start(ch + 1)
            pltpu.sync_copy(bufs[ch % 2],
                            out_hbm.at[pl.ds(r0 + ch * rc, rc)])

    return pl.kernel(
        body,
        mesh=plsc.VectorSubcoreMesh(core_axis_name="c", subcore_axis_name="s"),
        out_type=jax.ShapeDtypeStruct((N * K, d), jnp.float32),
        scratch_types=[
            pltpu.VMEM((_RPW,), jnp.int32),
            pltpu.VMEM((rc, d), jnp.float32),
            pltpu.VMEM((rc, d), jnp.float32),
            pltpu.SemaphoreType.DMA,
            pltpu.SemaphoreType.DMA,
        ],
    )


@functools.cache
def _scgather_call(d):
    return _make_scgather(d, 64 if d == D1 else 128)


def _scgather(X, idxf):
    return _scgather_call(X.shape[1])(X, idxf)


def kernel(Coordinate3D, Feature512D, CenterScore, W1, b1, W2, b2, W3, b3,
           W4, b4, W5, b5, W6, b6, Wo, bo):
    C = Coordinate3D[0]
    F = Feature512D[0]
    CS = CenterScore[0]

    Cp = jnp.pad(C, ((0, 0), (0, 128 - 3)))
    CSp = jnp.pad(CS, ((0, 0), (0, 64 - 1)))
    W1p = jnp.pad(W1, ((0, 128 - 3), (0, 0)))

    d_cat = 577
    Wa1 = jnp.pad(W3[:d_cat], ((0, D1 - d_cat), (0, 0)))
    Wb1 = jnp.pad(W3[d_cat:], ((0, D1 - d_cat), (0, 0)))
    Wa2, Wb2 = W4[:DM], W4[DM:]
    Wa3, Wb3 = W5[:DM], W5[DM:]

    b1r = b1.reshape(1, -1)
    b2r = b2.reshape(1, -1)
    b3r = b3.reshape(1, -1)
    b4r = b4.reshape(1, -1)
    b5r = b5.reshape(1, -1)
    b6r = b6.reshape(1, -1)
    WoT = Wo.reshape(1, -1)
    bor = bo.reshape(1, 1)

    X1 = _front(Cp, F, CSp, W1p, b1r, W2, b2r)

    A1 = _amm(X1, Wa1, D1)
    idx1 = _knn(X1, X1.T, D1)
    nb1 = _scgather(X1, idx1.reshape(-1))
    E1 = _edgemax(A1, nb1, X1, Wb1, b3r, D1)

    A2 = _amm(E1, Wa2, DM)
    idx2 = _knn(E1, E1.T, DM)
    nb2 = _scgather(E1, idx2.reshape(-1))
    E2, X3 = _edgemax_res(A2, nb2, E1, Wb2, b4r, E1, DM)

    A3 = _amm(X3, Wa3, DM)
    idx3 = _knn(X3, X3.T, DM)
    nb3 = _scgather(X3, idx3.reshape(-1))
    _, X4 = _edgemax_res(A3, nb3, X3, Wb3, b5r, E2, DM)

    return _head(X4, W6, b6r, WoT, bor).reshape(1, N, 1)
